# baseline (speedup 1.0000x reference)
import functools

import jax
import jax.numpy as jnp
import numpy as np
from jax.experimental import pallas as pl
from jax.experimental.pallas import tpu as pltpu

_STEPS = 5
_ROTS = ((13, 15, 26, 6), (17, 29, 16, 24))
_BLOCK_ROWS = 64


def _threefry_bits(e, seed):
    ks1 = np.uint32(seed)
    ks2 = np.uint32(np.uint32(seed) ^ np.uint32(0x1BD11BDA))
    ks = (np.uint32(0), ks1, ks2)
    x0 = jnp.zeros_like(e)
    x1 = e + ks1
    for i in range(5):
        for d in _ROTS[i % 2]:
            x0 = x0 + x1
            x1 = (x1 << jnp.uint32(d)) | (x1 >> jnp.uint32(32 - d))
            x1 = x1 ^ x0
        x0 = x0 + ks[(i + 1) % 3]
        x1 = x1 + np.uint32((int(ks[(i + 2) % 3]) + i + 1) & 0xFFFFFFFF)
    return x0 ^ x1


def _bits_to_uniform(bits):
    f = jax.lax.bitcast_convert_type(
        (bits >> jnp.uint32(9)) | jnp.uint32(0x3F800000), jnp.float32)
    return f - 1.0


def _schedules(sp):
    gamma_start = 0.1 * (1.0 - sp) + 0.001
    gamma_end = gamma_start * 0.1
    eps_start = 0.005 * sp + 0.0001
    eps_end = eps_start * 0.1
    g_cum, e_cum = [], []
    pg = jnp.float32(1.0)
    pe = jnp.float32(1.0)
    for s in range(_STEPS):
        frac = jnp.float32(s / (_STEPS - 1))
        g = gamma_start + (gamma_end - gamma_start) * frac
        ep = jnp.minimum(eps_start + (eps_end - eps_start) * frac,
                         jnp.float32(0.01))
        pg = pg * (1.0 - g)
        pe = pe * (1.0 - ep)
        g_cum.append(1.0 - pg)
        e_cum.append(1.0 - pe)
    return g_cum, e_cum


def _count_zero_body(x_ref, o_ref):
    @pl.when(pl.program_id(0) == 0)
    def _init():
        o_ref[0, 0] = jnp.float32(0.0)

    o_ref[0, 0] += jnp.sum((x_ref[:, :] == 0.0).astype(jnp.float32))


def _flip_body(sp_ref, t_ref, x_ref, o_ref, *, n_cols):
    sp = sp_ref[0, 0]
    g_cum, e_cum = _schedules(sp)

    t = t_ref[:, :]
    a0 = jnp.float32(0.0) * jnp.ones_like(t, dtype=jnp.float32)
    a1 = a0
    for s in range(_STEPS):
        sel = t == s
        a0 = jnp.where(sel, g_cum[s], a0)
        a1 = jnp.where(sel, e_cum[s], a1)

    br, w = x_ref.shape
    row0 = jnp.uint32(pl.program_id(0) * br)
    rows = jax.lax.broadcasted_iota(jnp.uint32, (br, w), 0) + row0
    cols = jax.lax.broadcasted_iota(jnp.uint32, (br, w), 1)
    e = rows * jnp.uint32(n_cols) + cols

    u_noise = _bits_to_uniform(_threefry_bits(e, 1234))
    u_bern = _bits_to_uniform(_threefry_bits(e, 5678))

    x = x_ref[:, :]
    a = jnp.where(x == 0.0, a0, a1)
    prob = jax.nn.sigmoid(a - u_noise)
    flip = u_bern < prob
    o_ref[:, :] = jnp.where(flip, 1.0 - x, x)


@jax.jit
def kernel(x_start, t):
    n_rows, n_cols = x_start.shape
    br = min(_BLOCK_ROWS, n_rows)
    grid = n_rows // br

    count0 = pl.pallas_call(
        _count_zero_body,
        grid=(grid,),
        in_specs=[pl.BlockSpec((br, n_cols), lambda i: (i, 0))],
        out_specs=pl.BlockSpec(memory_space=pltpu.SMEM),
        out_shape=jax.ShapeDtypeStruct((1, 1), jnp.float32),
    )(x_start)

    sparsity = count0 / jnp.float32(n_rows * n_cols)

    t2 = t.reshape(n_rows, 1)
    out = pl.pallas_call(
        functools.partial(_flip_body, n_cols=n_cols),
        grid=(grid,),
        in_specs=[
            pl.BlockSpec(memory_space=pltpu.SMEM),
            pl.BlockSpec((br, 1), lambda i: (i, 0)),
            pl.BlockSpec((br, n_cols), lambda i: (i, 0)),
        ],
        out_specs=pl.BlockSpec((br, n_cols), lambda i: (i, 0)),
        out_shape=jax.ShapeDtypeStruct((n_rows, n_cols), jnp.float32),
    )(sparsity, t2, x_start)
    return out


# speedup vs baseline: 1.9365x; 1.9365x over previous
import functools

import jax
import jax.numpy as jnp
import numpy as np
from jax.experimental import pallas as pl
from jax.experimental.pallas import tpu as pltpu

_STEPS = 5
_ROTS = ((13, 15, 26, 6), (17, 29, 16, 24))
_BLOCK_ROWS = 64
_CW = 1024


def _threefry_bits(e, seed):
    ks1 = np.uint32(seed)
    ks2 = np.uint32(np.uint32(seed) ^ np.uint32(0x1BD11BDA))
    ks = (np.uint32(0), ks1, ks2)
    x0 = jnp.zeros_like(e)
    x1 = e + ks1
    for i in range(5):
        for d in _ROTS[i % 2]:
            x0 = x0 + x1
            x1 = (x1 << jnp.uint32(d)) | (x1 >> jnp.uint32(32 - d))
            x1 = x1 ^ x0
        x0 = x0 + ks[(i + 1) % 3]
        x1 = x1 + np.uint32((int(ks[(i + 2) % 3]) + i + 1) & 0xFFFFFFFF)
    return x0 ^ x1


def _bits_to_uniform(bits):
    f = jax.lax.bitcast_convert_type(
        (bits >> jnp.uint32(9)) | jnp.uint32(0x3F800000), jnp.float32)
    return f - 1.0


def _schedules(sp):
    gamma_start = 0.1 * (1.0 - sp) + 0.001
    gamma_end = gamma_start * 0.1
    eps_start = 0.005 * sp + 0.0001
    eps_end = eps_start * 0.1
    g_cum, e_cum = [], []
    pg = jnp.float32(1.0)
    pe = jnp.float32(1.0)
    for s in range(_STEPS):
        frac = jnp.float32(s / (_STEPS - 1))
        g = gamma_start + (gamma_end - gamma_start) * frac
        ep = jnp.minimum(eps_start + (eps_end - eps_start) * frac,
                         jnp.float32(0.01))
        pg = pg * (1.0 - g)
        pe = pe * (1.0 - ep)
        g_cum.append(1.0 - pg)
        e_cum.append(1.0 - pe)
    return g_cum, e_cum


def _count_zero_body(x_ref, o_ref):
    @pl.when(pl.program_id(0) == 0)
    def _init():
        o_ref[0, 0] = jnp.float32(0.0)

    o_ref[0, 0] += jnp.sum((x_ref[:, :] == 0.0).astype(jnp.float32))


def _flip_chunk(x, e, a0c, a1c):
    u_noise = _bits_to_uniform(_threefry_bits(e, 1234))
    u_bern = _bits_to_uniform(_threefry_bits(e, 5678))
    a = jnp.where(x == 0.0, a0c, a1c)
    prob = jax.nn.sigmoid(a - u_noise)
    flip = u_bern < prob
    return jnp.where(flip, 1.0 - x, x)


def _flip_body(sp_ref, t_ref, x_ref, o_ref, *, n_cols):
    sp = sp_ref[0, 0]
    g_cum, e_cum = _schedules(sp)

    br, w = x_ref.shape
    n_full = w // _CW
    tail = w - n_full * _CW
    row0 = jnp.uint32(pl.program_id(0) * br)

    pat = (jax.lax.broadcasted_iota(jnp.uint32, (8, _CW), 0)
           * jnp.uint32(n_cols)
           + jax.lax.broadcasted_iota(jnp.uint32, (8, _CW), 1))
    if tail:
        pat_t = (jax.lax.broadcasted_iota(jnp.uint32, (8, tail), 0)
                 * jnp.uint32(n_cols)
                 + jax.lax.broadcasted_iota(jnp.uint32, (8, tail), 1))

    def row_group(ri, _):
        rs = ri * 8
        tc = t_ref[pl.ds(rs, 8), :]
        a0c = jnp.zeros_like(tc, dtype=jnp.float32)
        a1c = a0c
        for s in range(_STEPS):
            sel = tc == s
            a0c = jnp.where(sel, g_cum[s], a0c)
            a1c = jnp.where(sel, e_cum[s], a1c)
        ebase = (row0 + jnp.uint32(rs)) * jnp.uint32(n_cols)

        def col_chunk(ci, _):
            cs = ci * _CW
            x = x_ref[pl.ds(rs, 8), pl.ds(cs, _CW)]
            e = pat + (ebase + jnp.uint32(cs))
            o_ref[pl.ds(rs, 8), pl.ds(cs, _CW)] = _flip_chunk(x, e, a0c, a1c)
            return 0

        jax.lax.fori_loop(0, n_full, col_chunk, 0)
        if tail:
            x = x_ref[pl.ds(rs, 8), pl.ds(n_full * _CW, tail)]
            e = pat_t + (ebase + jnp.uint32(n_full * _CW))
            o_ref[pl.ds(rs, 8), pl.ds(n_full * _CW, tail)] = _flip_chunk(
                x, e, a0c, a1c)
        return 0

    jax.lax.fori_loop(0, br // 8, row_group, 0)


@jax.jit
def kernel(x_start, t):
    n_rows, n_cols = x_start.shape
    br = min(_BLOCK_ROWS, n_rows)
    grid = n_rows // br

    count0 = pl.pallas_call(
        _count_zero_body,
        grid=(grid,),
        in_specs=[pl.BlockSpec((br, n_cols), lambda i: (i, 0))],
        out_specs=pl.BlockSpec(memory_space=pltpu.SMEM),
        out_shape=jax.ShapeDtypeStruct((1, 1), jnp.float32),
    )(x_start)

    sparsity = count0 / jnp.float32(n_rows * n_cols)

    t2 = t.reshape(n_rows, 1)
    out = pl.pallas_call(
        functools.partial(_flip_body, n_cols=n_cols),
        grid=(grid,),
        in_specs=[
            pl.BlockSpec(memory_space=pltpu.SMEM),
            pl.BlockSpec((br, 1), lambda i: (i, 0)),
            pl.BlockSpec((br, n_cols), lambda i: (i, 0)),
        ],
        out_specs=pl.BlockSpec((br, n_cols), lambda i: (i, 0)),
        out_shape=jax.ShapeDtypeStruct((n_rows, n_cols), jnp.float32),
    )(sparsity, t2, x_start)
    return out


# speedup vs baseline: 1.9444x; 1.0041x over previous
import functools

import jax
import jax.numpy as jnp
import numpy as np
from jax.experimental import pallas as pl
from jax.experimental.pallas import tpu as pltpu

_STEPS = 5
_ROTS = ((13, 15, 26, 6), (17, 29, 16, 24))
_BLOCK_ROWS = 64
_CW = 1024


def _col_chunks(w):
    chunks = [(c, _CW) for c in range(0, w - _CW + 1, _CW)]
    done = len(chunks) * _CW
    if done < w:
        chunks.append((done, w - done))
    return chunks


def _threefry_bits(e, seed):
    ks1 = np.uint32(seed)
    ks2 = np.uint32(np.uint32(seed) ^ np.uint32(0x1BD11BDA))
    ks = (np.uint32(0), ks1, ks2)
    x0 = jnp.zeros_like(e)
    x1 = e + ks1
    for i in range(5):
        for d in _ROTS[i % 2]:
            x0 = x0 + x1
            x1 = (x1 << jnp.uint32(d)) | (x1 >> jnp.uint32(32 - d))
            x1 = x1 ^ x0
        x0 = x0 + ks[(i + 1) % 3]
        x1 = x1 + np.uint32((int(ks[(i + 2) % 3]) + i + 1) & 0xFFFFFFFF)
    return x0 ^ x1


def _bits_to_uniform(bits):
    f = jax.lax.bitcast_convert_type(
        (bits >> jnp.uint32(9)) | jnp.uint32(0x3F800000), jnp.float32)
    return f - 1.0


def _schedules(sp):
    gamma_start = 0.1 * (1.0 - sp) + 0.001
    gamma_end = gamma_start * 0.1
    eps_start = 0.005 * sp + 0.0001
    eps_end = eps_start * 0.1
    g_cum, e_cum = [], []
    pg = jnp.float32(1.0)
    pe = jnp.float32(1.0)
    for s in range(_STEPS):
        frac = jnp.float32(s / (_STEPS - 1))
        g = gamma_start + (gamma_end - gamma_start) * frac
        ep = jnp.minimum(eps_start + (eps_end - eps_start) * frac,
                         jnp.float32(0.01))
        pg = pg * (1.0 - g)
        pe = pe * (1.0 - ep)
        g_cum.append(1.0 - pg)
        e_cum.append(1.0 - pe)
    return g_cum, e_cum


def _count_zero_body(x_ref, o_ref):
    br, w = x_ref.shape
    chunks = _col_chunks(w)

    def row_group(ri, carry):
        rs = ri * 8
        s = carry
        for cs, cw in chunks:
            x = x_ref[pl.ds(rs, 8), pl.ds(cs, cw)]
            s += jnp.sum((x == 0.0).astype(jnp.float32))
        return s

    total = jax.lax.fori_loop(0, br // 8, row_group, jnp.float32(0.0))

    @pl.when(pl.program_id(0) == 0)
    def _init():
        o_ref[0, 0] = jnp.float32(0.0)

    o_ref[0, 0] += total


def _flip_chunk(x, e, a0c, a1c):
    u_noise = _bits_to_uniform(_threefry_bits(e, 1234))
    u_bern = _bits_to_uniform(_threefry_bits(e, 5678))
    a = jnp.where(x == 0.0, a0c, a1c)
    prob = jax.nn.sigmoid(a - u_noise)
    flip = u_bern < prob
    return jnp.where(flip, 1.0 - x, x)


def _flip_body(sp_ref, t_ref, x_ref, o_ref, *, n_cols):
    sp = sp_ref[0, 0]
    g_cum, e_cum = _schedules(sp)

    br, w = x_ref.shape
    chunks = _col_chunks(w)
    row0 = jnp.uint32(pl.program_id(0) * br)

    pats = {}
    for _, cw in chunks:
        if cw not in pats:
            pats[cw] = (jax.lax.broadcasted_iota(jnp.uint32, (8, cw), 0)
                        * jnp.uint32(n_cols)
                        + jax.lax.broadcasted_iota(jnp.uint32, (8, cw), 1))

    def row_group(ri, _):
        rs = ri * 8
        tc = t_ref[pl.ds(rs, 8), :]
        a0c = jnp.zeros_like(tc, dtype=jnp.float32)
        a1c = a0c
        for s in range(_STEPS):
            sel = tc == s
            a0c = jnp.where(sel, g_cum[s], a0c)
            a1c = jnp.where(sel, e_cum[s], a1c)
        ebase = (row0 + jnp.uint32(rs)) * jnp.uint32(n_cols)

        for cs, cw in chunks:
            x = x_ref[pl.ds(rs, 8), pl.ds(cs, cw)]
            e = pats[cw] + (ebase + jnp.uint32(cs))
            o_ref[pl.ds(rs, 8), pl.ds(cs, cw)] = _flip_chunk(x, e, a0c, a1c)
        return 0

    jax.lax.fori_loop(0, br // 8, row_group, 0)


@jax.jit
def kernel(x_start, t):
    n_rows, n_cols = x_start.shape
    br = min(_BLOCK_ROWS, n_rows)
    grid = n_rows // br

    count0 = pl.pallas_call(
        _count_zero_body,
        grid=(grid,),
        in_specs=[pl.BlockSpec((br, n_cols), lambda i: (i, 0))],
        out_specs=pl.BlockSpec(memory_space=pltpu.SMEM),
        out_shape=jax.ShapeDtypeStruct((1, 1), jnp.float32),
    )(x_start)

    sparsity = count0 / jnp.float32(n_rows * n_cols)

    t2 = t.reshape(n_rows, 1)
    out = pl.pallas_call(
        functools.partial(_flip_body, n_cols=n_cols),
        grid=(grid,),
        in_specs=[
            pl.BlockSpec(memory_space=pltpu.SMEM),
            pl.BlockSpec((br, 1), lambda i: (i, 0)),
            pl.BlockSpec((br, n_cols), lambda i: (i, 0)),
        ],
        out_specs=pl.BlockSpec((br, n_cols), lambda i: (i, 0)),
        out_shape=jax.ShapeDtypeStruct((n_rows, n_cols), jnp.float32),
    )(sparsity, t2, x_start)
    return out


# speedup vs baseline: 1.9761x; 1.0163x over previous
import functools

import jax
import jax.numpy as jnp
import numpy as np
from jax.experimental import pallas as pl
from jax.experimental.pallas import tpu as pltpu

_STEPS = 5
_ROTS = ((13, 15, 26, 6), (17, 29, 16, 24))
_BLOCK_ROWS = 64
_CW = 1024


def _col_chunks(w):
    chunks = [(c, _CW) for c in range(0, w - _CW + 1, _CW)]
    done = len(chunks) * _CW
    if done < w:
        chunks.append((done, w - done))
    return chunks


def _threefry_bits(x1, seed):
    ks1 = np.uint32(seed)
    ks2 = np.uint32(np.uint32(seed) ^ np.uint32(0x1BD11BDA))
    ks = (np.uint32(0), ks1, ks2)
    x0 = jnp.zeros_like(x1)
    for i in range(5):
        for d in _ROTS[i % 2]:
            x0 = x0 + x1
            x1 = (x1 << jnp.uint32(d)) | (x1 >> jnp.uint32(32 - d))
            x1 = x1 ^ x0
        x0 = x0 + ks[(i + 1) % 3]
        x1 = x1 + np.uint32((int(ks[(i + 2) % 3]) + i + 1) & 0xFFFFFFFF)
    return x0 ^ x1


def _bits_to_uniform(bits):
    f = jax.lax.bitcast_convert_type(
        (bits >> jnp.uint32(9)) | jnp.uint32(0x3F800000), jnp.float32)
    return f - 1.0


def _schedules(sp):
    gamma_start = 0.1 * (1.0 - sp) + 0.001
    gamma_end = gamma_start * 0.1
    eps_start = 0.005 * sp + 0.0001
    eps_end = eps_start * 0.1
    g_cum, e_cum = [], []
    pg = jnp.float32(1.0)
    pe = jnp.float32(1.0)
    for s in range(_STEPS):
        frac = jnp.float32(s / (_STEPS - 1))
        g = gamma_start + (gamma_end - gamma_start) * frac
        ep = jnp.minimum(eps_start + (eps_end - eps_start) * frac,
                         jnp.float32(0.01))
        pg = pg * (1.0 - g)
        pe = pe * (1.0 - ep)
        g_cum.append(1.0 - pg)
        e_cum.append(1.0 - pe)
    return g_cum, e_cum


def _count_zero_body(x_ref, o_ref):
    br, w = x_ref.shape
    chunks = _col_chunks(w)

    def row_group(ri, carry):
        rs = ri * 8
        accs = []
        for (cs, cw), acc in zip(chunks, carry):
            x = x_ref[pl.ds(rs, 8), pl.ds(cs, cw)]
            accs.append(acc + (x == 0.0).astype(jnp.float32))
        return tuple(accs)

    init = tuple(jnp.zeros((8, cw), jnp.float32) for _, cw in chunks)
    accs = jax.lax.fori_loop(0, br // 8, row_group, init)
    total = jnp.float32(0.0)
    for acc in accs:
        total += jnp.sum(acc)

    @pl.when(pl.program_id(0) == 0)
    def _init():
        o_ref[0, 0] = jnp.float32(0.0)

    o_ref[0, 0] += total


def _flip_chunk(x, x1a, x1b, a0c, a1c):
    u_noise = _bits_to_uniform(_threefry_bits(x1a, 1234))
    u_bern = _bits_to_uniform(_threefry_bits(x1b, 5678))
    a = jnp.where(x == 0.0, a0c, a1c)
    prob = jax.nn.sigmoid(a - u_noise)
    flip = u_bern < prob
    return jnp.where(flip, 1.0 - x, x)


def _flip_body(sp_ref, t_ref, pa_ref, pb_ref, x_ref, o_ref, *, n_cols):
    sp = sp_ref[0, 0]
    g_cum, e_cum = _schedules(sp)

    br, w = x_ref.shape
    chunks = _col_chunks(w)
    row0 = jnp.uint32(pl.program_id(0) * br)

    def row_group(ri, _):
        rs = ri * 8
        tc = t_ref[pl.ds(rs, 8), :]
        a0c = jnp.zeros_like(tc, dtype=jnp.float32)
        a1c = a0c
        for s in range(_STEPS):
            sel = tc == s
            a0c = jnp.where(sel, g_cum[s], a0c)
            a1c = jnp.where(sel, e_cum[s], a1c)
        ebase = (row0 + jnp.uint32(rs)) * jnp.uint32(n_cols)

        for cs, cw in chunks:
            x = x_ref[pl.ds(rs, 8), pl.ds(cs, cw)]
            x1a = pa_ref[:, pl.ds(cs, cw)] + ebase
            x1b = pb_ref[:, pl.ds(cs, cw)] + ebase
            o_ref[pl.ds(rs, 8), pl.ds(cs, cw)] = _flip_chunk(
                x, x1a, x1b, a0c, a1c)
        return 0

    jax.lax.fori_loop(0, br // 8, row_group, 0)


@jax.jit
def kernel(x_start, t):
    n_rows, n_cols = x_start.shape
    br = min(_BLOCK_ROWS, n_rows)
    grid = n_rows // br

    count0 = pl.pallas_call(
        _count_zero_body,
        grid=(grid,),
        in_specs=[pl.BlockSpec((br, n_cols), lambda i: (i, 0))],
        out_specs=pl.BlockSpec(memory_space=pltpu.SMEM),
        out_shape=jax.ShapeDtypeStruct((1, 1), jnp.float32),
    )(x_start)

    sparsity = count0 / jnp.float32(n_rows * n_cols)

    t2 = t.reshape(n_rows, 1)
    pat = (jnp.arange(8, dtype=jnp.uint32)[:, None] * jnp.uint32(n_cols)
           + jnp.arange(n_cols, dtype=jnp.uint32)[None, :])
    pa = pat + jnp.uint32(1234)
    pb = pat + jnp.uint32(5678)
    out = pl.pallas_call(
        functools.partial(_flip_body, n_cols=n_cols),
        grid=(grid,),
        in_specs=[
            pl.BlockSpec(memory_space=pltpu.SMEM),
            pl.BlockSpec((br, 1), lambda i: (i, 0)),
            pl.BlockSpec((8, n_cols), lambda i: (0, 0)),
            pl.BlockSpec((8, n_cols), lambda i: (0, 0)),
            pl.BlockSpec((br, n_cols), lambda i: (i, 0)),
        ],
        out_specs=pl.BlockSpec((br, n_cols), lambda i: (i, 0)),
        out_shape=jax.ShapeDtypeStruct((n_rows, n_cols), jnp.float32),
        compiler_params=pltpu.CompilerParams(
            dimension_semantics=("parallel",)),
    )(sparsity, t2, pa, pb, x_start)
    return out


# speedup vs baseline: 2.3658x; 1.1972x over previous
import functools

import jax
import jax.numpy as jnp
import numpy as np
from jax.experimental import pallas as pl
from jax.experimental.pallas import tpu as pltpu

_STEPS = 5
_ROTS = ((13, 15, 26, 6), (17, 29, 16, 24))
_BLOCK_ROWS = 200
_CW = 1024


def _col_chunks(w):
    chunks = [(c, _CW) for c in range(0, w - _CW + 1, _CW)]
    done = len(chunks) * _CW
    if done < w:
        chunks.append((done, w - done))
    return chunks


def _threefry_bits(x1, seed):
    ks1 = np.uint32(seed)
    ks2 = np.uint32(np.uint32(seed) ^ np.uint32(0x1BD11BDA))
    ks = (np.uint32(0), ks1, ks2)
    x0 = jnp.zeros_like(x1)
    for i in range(5):
        for d in _ROTS[i % 2]:
            x0 = x0 + x1
            x1 = (x1 << jnp.uint32(d)) | (x1 >> jnp.uint32(32 - d))
            x1 = x1 ^ x0
        x0 = x0 + ks[(i + 1) % 3]
        x1 = x1 + np.uint32((int(ks[(i + 2) % 3]) + i + 1) & 0xFFFFFFFF)
    return x0 ^ x1


def _bits_to_uniform(bits):
    f = jax.lax.bitcast_convert_type(
        (bits >> jnp.uint32(9)) | jnp.uint32(0x3F800000), jnp.float32)
    return f - 1.0


def _schedules(sp):
    gamma_start = 0.1 * (1.0 - sp) + 0.001
    gamma_end = gamma_start * 0.1
    eps_start = 0.005 * sp + 0.0001
    eps_end = eps_start * 0.1
    g_cum, e_cum = [], []
    pg = jnp.float32(1.0)
    pe = jnp.float32(1.0)
    for s in range(_STEPS):
        frac = jnp.float32(s / (_STEPS - 1))
        g = gamma_start + (gamma_end - gamma_start) * frac
        ep = jnp.minimum(eps_start + (eps_end - eps_start) * frac,
                         jnp.float32(0.01))
        pg = pg * (1.0 - g)
        pe = pe * (1.0 - ep)
        g_cum.append(1.0 - pg)
        e_cum.append(1.0 - pe)
    return g_cum, e_cum


def _count_zero_body(x_ref, o_ref):
    br, w = x_ref.shape
    chunks = _col_chunks(w)
    groups = br // 8
    unroll = 5 if groups % 5 == 0 else (4 if groups % 4 == 0 else 1)

    def row_groups(gi, carry):
        accs = list(carry)
        for u in range(unroll):
            rs = (gi * unroll + u) * 8
            for k, (cs, cw) in enumerate(chunks):
                accs[k] = accs[k] + x_ref[pl.ds(rs, 8), pl.ds(cs, cw)]
        return tuple(accs)

    init = tuple(jnp.zeros((8, cw), jnp.float32) for _, cw in chunks)
    accs = jax.lax.fori_loop(0, groups // unroll, row_groups, init)
    ones = jnp.float32(0.0)
    for acc in accs:
        ones += jnp.sum(acc)
    total = jnp.float32(br * w) - ones

    @pl.when(pl.program_id(0) == 0)
    def _init():
        o_ref[0, 0] = jnp.float32(0.0)

    o_ref[0, 0] += total


def _flip_chunk(x, x1a, x1b, a0c, a1c):
    u_noise = _bits_to_uniform(_threefry_bits(x1a, 1234))
    u_bern = _bits_to_uniform(_threefry_bits(x1b, 5678))
    a = jnp.where(x == 0.0, a0c, a1c)
    prob = jax.nn.sigmoid(a - u_noise)
    flip = u_bern < prob
    return jnp.where(flip, 1.0 - x, x)


def _flip_body(sp_ref, t_ref, pa_ref, pb_ref, x_ref, o_ref, *, n_cols):
    sp = sp_ref[0, 0]
    g_cum, e_cum = _schedules(sp)

    br, w = x_ref.shape
    chunks = _col_chunks(w)
    col0 = jnp.uint32(pl.program_id(0) * br)

    tc = t_ref[0:1, :]
    a0v = jnp.zeros_like(tc, dtype=jnp.float32)
    a1v = a0v
    for s in range(_STEPS):
        sel = tc == s
        a0v = jnp.where(sel, g_cum[s], a0v)
        a1v = jnp.where(sel, e_cum[s], a1v)

    def row_group(ri, _):
        rs = ri * 8
        cbase = col0 + jnp.uint32(rs)
        for cs, cw in chunks:
            x = x_ref[pl.ds(rs, 8), pl.ds(cs, cw)]
            x1a = pa_ref[:, pl.ds(cs, cw)] + cbase
            x1b = pb_ref[:, pl.ds(cs, cw)] + cbase
            o_ref[pl.ds(rs, 8), pl.ds(cs, cw)] = _flip_chunk(
                x, x1a, x1b, a0v[:, cs:cs + cw], a1v[:, cs:cs + cw])
        return 0

    jax.lax.fori_loop(0, br // 8, row_group, 0)


@jax.jit
def kernel(x_start, t):
    n_rows, n_cols = x_start.shape
    xt = x_start.T

    brt = min(_BLOCK_ROWS, n_cols)
    grid = n_cols // brt

    count0 = pl.pallas_call(
        _count_zero_body,
        grid=(grid,),
        in_specs=[pl.BlockSpec((brt, n_rows), lambda i: (i, 0))],
        out_specs=pl.BlockSpec(memory_space=pltpu.SMEM),
        out_shape=jax.ShapeDtypeStruct((1, 1), jnp.float32),
    )(xt)

    sparsity = count0 / jnp.float32(n_rows * n_cols)

    t2 = t.reshape(1, n_rows)
    pat = (jnp.arange(n_rows, dtype=jnp.uint32)[None, :] * jnp.uint32(n_cols)
           + jnp.arange(8, dtype=jnp.uint32)[:, None])
    pa = pat + jnp.uint32(1234)
    pb = pat + jnp.uint32(5678)
    out_t = pl.pallas_call(
        functools.partial(_flip_body, n_cols=n_cols),
        grid=(grid,),
        in_specs=[
            pl.BlockSpec(memory_space=pltpu.SMEM),
            pl.BlockSpec((1, n_rows), lambda i: (0, 0)),
            pl.BlockSpec((8, n_rows), lambda i: (0, 0)),
            pl.BlockSpec((8, n_rows), lambda i: (0, 0)),
            pl.BlockSpec((brt, n_rows), lambda i: (i, 0)),
        ],
        out_specs=pl.BlockSpec((brt, n_rows), lambda i: (i, 0)),
        out_shape=jax.ShapeDtypeStruct((n_cols, n_rows), jnp.float32),
        compiler_params=pltpu.CompilerParams(
            dimension_semantics=("parallel",)),
    )(sparsity, t2, pa, pb, xt)
    return out_t.T


# speedup vs baseline: 2.4362x; 1.0297x over previous
import functools

import jax
import jax.numpy as jnp
import numpy as np
from jax.experimental import pallas as pl
from jax.experimental.pallas import tpu as pltpu

_STEPS = 5
_ROTS = ((13, 15, 26, 6), (17, 29, 16, 24))
_BLOCK_ROWS = 400
_CW = 1024


def _col_chunks(w):
    chunks = [(c, _CW) for c in range(0, w - _CW + 1, _CW)]
    done = len(chunks) * _CW
    if done < w:
        chunks.append((done, w - done))
    return chunks


def _threefry_bits(x1, seed):
    ks1 = np.uint32(seed)
    ks2 = np.uint32(np.uint32(seed) ^ np.uint32(0x1BD11BDA))
    ks = (np.uint32(0), ks1, ks2)
    x0 = jnp.zeros_like(x1)
    for i in range(5):
        for d in _ROTS[i % 2]:
            x0 = x0 + x1
            x1 = (x1 << jnp.uint32(d)) | (x1 >> jnp.uint32(32 - d))
            x1 = x1 ^ x0
        x0 = x0 + ks[(i + 1) % 3]
        x1 = x1 + np.uint32((int(ks[(i + 2) % 3]) + i + 1) & 0xFFFFFFFF)
    return x0 ^ x1


def _bits_to_uniform(bits):
    f = jax.lax.bitcast_convert_type(
        (bits >> jnp.uint32(9)) | jnp.uint32(0x3F800000), jnp.float32)
    return f - 1.0


def _schedules(sp):
    gamma_start = 0.1 * (1.0 - sp) + 0.001
    gamma_end = gamma_start * 0.1
    eps_start = 0.005 * sp + 0.0001
    eps_end = eps_start * 0.1
    g_cum, e_cum = [], []
    pg = jnp.float32(1.0)
    pe = jnp.float32(1.0)
    for s in range(_STEPS):
        frac = jnp.float32(s / (_STEPS - 1))
        g = gamma_start + (gamma_end - gamma_start) * frac
        ep = jnp.minimum(eps_start + (eps_end - eps_start) * frac,
                         jnp.float32(0.01))
        pg = pg * (1.0 - g)
        pe = pe * (1.0 - ep)
        g_cum.append(1.0 - pg)
        e_cum.append(1.0 - pe)
    return g_cum, e_cum


def _count_zero_body(x_ref, o_ref):
    br, w = x_ref.shape
    chunks = _col_chunks(w)
    groups = br // 8
    unroll = 5 if groups % 5 == 0 else (4 if groups % 4 == 0 else 1)

    def row_groups(gi, carry):
        accs = list(carry)
        for u in range(unroll):
            rs = (gi * unroll + u) * 8
            for k, (cs, cw) in enumerate(chunks):
                accs[k] = accs[k] + x_ref[pl.ds(rs, 8), pl.ds(cs, cw)]
        return tuple(accs)

    init = tuple(jnp.zeros((8, cw), jnp.float32) for _, cw in chunks)
    accs = jax.lax.fori_loop(0, groups // unroll, row_groups, init)
    ones = jnp.float32(0.0)
    for acc in accs:
        ones += jnp.sum(acc)
    total = jnp.float32(br * w) - ones

    @pl.when(pl.program_id(0) == 0)
    def _init():
        o_ref[0, 0] = jnp.float32(0.0)

    o_ref[0, 0] += total


def _flip_chunk(x, x1a, x1b, a0c, a1c):
    u_noise = _bits_to_uniform(_threefry_bits(x1a, 1234))
    u_bern = _bits_to_uniform(_threefry_bits(x1b, 5678))
    a = jnp.where(x == 0.0, a0c, a1c)
    prob = jax.nn.sigmoid(a - u_noise)
    flip = u_bern < prob
    return jnp.where(flip, 1.0 - x, x)


def _flip_body(sp_ref, t_ref, pa_ref, pb_ref, x_ref, o_ref, *, n_cols):
    sp = sp_ref[0, 0]
    g_cum, e_cum = _schedules(sp)

    br, w = x_ref.shape
    chunks = _col_chunks(w)
    col0 = jnp.uint32(pl.program_id(0) * br)

    tc = t_ref[0:1, :]
    a0v = jnp.zeros_like(tc, dtype=jnp.float32)
    a1v = a0v
    for s in range(_STEPS):
        sel = tc == s
        a0v = jnp.where(sel, g_cum[s], a0v)
        a1v = jnp.where(sel, e_cum[s], a1v)

    groups = br // 8
    unroll = 2 if groups % 2 == 0 else 1

    def row_groups(gi, _):
        for u in range(unroll):
            rs = (gi * unroll + u) * 8
            cbase = col0 + jnp.uint32(rs)
            for cs, cw in chunks:
                x = x_ref[pl.ds(rs, 8), pl.ds(cs, cw)]
                x1a = pa_ref[:, pl.ds(cs, cw)] + cbase
                x1b = pb_ref[:, pl.ds(cs, cw)] + cbase
                o_ref[pl.ds(rs, 8), pl.ds(cs, cw)] = _flip_chunk(
                    x, x1a, x1b, a0v[:, cs:cs + cw], a1v[:, cs:cs + cw])
        return 0

    jax.lax.fori_loop(0, groups // unroll, row_groups, 0)


@jax.jit
def kernel(x_start, t):
    n_rows, n_cols = x_start.shape
    xt = x_start.T

    brt = min(_BLOCK_ROWS, n_cols)
    grid = n_cols // brt

    count0 = pl.pallas_call(
        _count_zero_body,
        grid=(grid,),
        in_specs=[pl.BlockSpec((brt, n_rows), lambda i: (i, 0))],
        out_specs=pl.BlockSpec(memory_space=pltpu.SMEM),
        out_shape=jax.ShapeDtypeStruct((1, 1), jnp.float32),
    )(xt)

    sparsity = count0 / jnp.float32(n_rows * n_cols)

    t2 = t.reshape(1, n_rows)
    pat = (jnp.arange(n_rows, dtype=jnp.uint32)[None, :] * jnp.uint32(n_cols)
           + jnp.arange(8, dtype=jnp.uint32)[:, None])
    pa = pat + jnp.uint32(1234)
    pb = pat + jnp.uint32(5678)
    out_t = pl.pallas_call(
        functools.partial(_flip_body, n_cols=n_cols),
        grid=(grid,),
        in_specs=[
            pl.BlockSpec(memory_space=pltpu.SMEM),
            pl.BlockSpec((1, n_rows), lambda i: (0, 0)),
            pl.BlockSpec((8, n_rows), lambda i: (0, 0)),
            pl.BlockSpec((8, n_rows), lambda i: (0, 0)),
            pl.BlockSpec((brt, n_rows), lambda i: (i, 0)),
        ],
        out_specs=pl.BlockSpec((brt, n_rows), lambda i: (i, 0)),
        out_shape=jax.ShapeDtypeStruct((n_cols, n_rows), jnp.float32),
        compiler_params=pltpu.CompilerParams(
            dimension_semantics=("parallel",)),
    )(sparsity, t2, pa, pb, xt)
    return out_t.T


# speedup vs baseline: 2.4364x; 1.0001x over previous
import functools

import jax
import jax.numpy as jnp
import numpy as np
from jax.experimental import pallas as pl
from jax.experimental.pallas import tpu as pltpu

_STEPS = 5
_ROTS = ((13, 15, 26, 6), (17, 29, 16, 24))
_BLOCK_ROWS = 400
_CW = 1024


def _col_chunks(w):
    chunks = [(c, _CW) for c in range(0, w - _CW + 1, _CW)]
    done = len(chunks) * _CW
    if done < w:
        chunks.append((done, w - done))
    return chunks


def _threefry_bits(x1, seed):
    ks1 = np.uint32(seed)
    ks2 = np.uint32(np.uint32(seed) ^ np.uint32(0x1BD11BDA))
    ks = (np.uint32(0), ks1, ks2)
    x0 = jnp.zeros_like(x1)
    for i in range(5):
        for d in _ROTS[i % 2]:
            x0 = x0 + x1
            x1 = (x1 << jnp.uint32(d)) | (x1 >> jnp.uint32(32 - d))
            x1 = x1 ^ x0
        x0 = x0 + ks[(i + 1) % 3]
        x1 = x1 + np.uint32((int(ks[(i + 2) % 3]) + i + 1) & 0xFFFFFFFF)
    return x0 ^ x1


def _bits_to_uniform(bits):
    f = jax.lax.bitcast_convert_type(
        (bits >> jnp.uint32(9)) | jnp.uint32(0x3F800000), jnp.float32)
    return f - 1.0


def _schedules(sp):
    gamma_start = 0.1 * (1.0 - sp) + 0.001
    gamma_end = gamma_start * 0.1
    eps_start = 0.005 * sp + 0.0001
    eps_end = eps_start * 0.1
    g_cum, e_cum = [], []
    pg = jnp.float32(1.0)
    pe = jnp.float32(1.0)
    for s in range(_STEPS):
        frac = jnp.float32(s / (_STEPS - 1))
        g = gamma_start + (gamma_end - gamma_start) * frac
        ep = jnp.minimum(eps_start + (eps_end - eps_start) * frac,
                         jnp.float32(0.01))
        pg = pg * (1.0 - g)
        pe = pe * (1.0 - ep)
        g_cum.append(1.0 - pg)
        e_cum.append(1.0 - pe)
    return g_cum, e_cum


def _count_zero_body(xa_ref, xb_ref, o_ref):
    br, w = xa_ref.shape
    chunks = _col_chunks(w)
    groups = br // 8
    unroll = 5 if groups % 5 == 0 else (4 if groups % 4 == 0 else 1)

    def row_groups(gi, carry):
        accs = list(carry)
        for u in range(unroll):
            rs = (gi * unroll + u) * 8
            for k, (cs, cw) in enumerate(chunks):
                accs[2 * k] = accs[2 * k] + xa_ref[pl.ds(rs, 8),
                                                   pl.ds(cs, cw)]
                accs[2 * k + 1] = accs[2 * k + 1] + xb_ref[pl.ds(rs, 8),
                                                           pl.ds(cs, cw)]
        return tuple(accs)

    init = tuple(jnp.zeros((8, cw), jnp.float32)
                 for _, cw in chunks for _ in range(2))
    accs = jax.lax.fori_loop(0, groups // unroll, row_groups, init)
    ones = jnp.float32(0.0)
    for acc in accs:
        ones += jnp.sum(acc)
    total = jnp.float32(2 * br * w) - ones

    @pl.when(pl.program_id(0) == 0)
    def _init():
        o_ref[0, 0] = jnp.float32(0.0)

    o_ref[0, 0] += total


def _flip_chunk(x, x1a, x1b, a0c, a1c):
    u_noise = _bits_to_uniform(_threefry_bits(x1a, 1234))
    u_bern = _bits_to_uniform(_threefry_bits(x1b, 5678))
    a = jnp.where(x == 0.0, a0c, a1c)
    prob = jax.nn.sigmoid(a - u_noise)
    flip = u_bern < prob
    return jnp.where(flip, 1.0 - x, x)


def _flip_body(sp_ref, t_ref, pa_ref, pb_ref, x_ref, o_ref, *, n_cols):
    sp = sp_ref[0, 0]
    g_cum, e_cum = _schedules(sp)

    br, w = x_ref.shape
    chunks = _col_chunks(w)
    col0 = jnp.uint32(pl.program_id(0) * br)

    tc = t_ref[0:1, :]
    a0v = jnp.zeros_like(tc, dtype=jnp.float32)
    a1v = a0v
    for s in range(_STEPS):
        sel = tc == s
        a0v = jnp.where(sel, g_cum[s], a0v)
        a1v = jnp.where(sel, e_cum[s], a1v)

    groups = br // 8
    unroll = 2 if groups % 2 == 0 else 1

    def row_groups(gi, _):
        for u in range(unroll):
            rs = (gi * unroll + u) * 8
            cbase = col0 + jnp.uint32(rs)
            for cs, cw in chunks:
                x = x_ref[pl.ds(rs, 8), pl.ds(cs, cw)]
                x1a = pa_ref[:, pl.ds(cs, cw)] + cbase
                x1b = pb_ref[:, pl.ds(cs, cw)] + cbase
                o_ref[pl.ds(rs, 8), pl.ds(cs, cw)] = _flip_chunk(
                    x, x1a, x1b, a0v[:, cs:cs + cw], a1v[:, cs:cs + cw])
        return 0

    jax.lax.fori_loop(0, groups // unroll, row_groups, 0)


@jax.jit
def kernel(x_start, t):
    n_rows, n_cols = x_start.shape
    xt = x_start.T

    brt = min(_BLOCK_ROWS, n_cols)
    grid = n_cols // brt

    cbr = brt // 2
    cgrid = n_cols // (2 * cbr)
    count0 = pl.pallas_call(
        _count_zero_body,
        grid=(cgrid,),
        in_specs=[
            pl.BlockSpec((cbr, n_rows), lambda i: (i, 0)),
            pl.BlockSpec((cbr, n_rows), lambda i, _g=cgrid: (i + _g, 0)),
        ],
        out_specs=pl.BlockSpec(memory_space=pltpu.SMEM),
        out_shape=jax.ShapeDtypeStruct((1, 1), jnp.float32),
    )(xt, xt)

    sparsity = count0 / jnp.float32(n_rows * n_cols)

    t2 = t.reshape(1, n_rows)
    pat = (jnp.arange(n_rows, dtype=jnp.uint32)[None, :] * jnp.uint32(n_cols)
           + jnp.arange(8, dtype=jnp.uint32)[:, None])
    pa = pat + jnp.uint32(1234)
    pb = pat + jnp.uint32(5678)
    out_t = pl.pallas_call(
        functools.partial(_flip_body, n_cols=n_cols),
        grid=(grid,),
        in_specs=[
            pl.BlockSpec(memory_space=pltpu.SMEM),
            pl.BlockSpec((1, n_rows), lambda i: (0, 0)),
            pl.BlockSpec((8, n_rows), lambda i: (0, 0)),
            pl.BlockSpec((8, n_rows), lambda i: (0, 0)),
            pl.BlockSpec((brt, n_rows), lambda i: (i, 0)),
        ],
        out_specs=pl.BlockSpec((brt, n_rows), lambda i: (i, 0)),
        out_shape=jax.ShapeDtypeStruct((n_cols, n_rows), jnp.float32),
        compiler_params=pltpu.CompilerParams(
            dimension_semantics=("parallel",)),
    )(sparsity, t2, pa, pb, xt)
    return out_t.T


# speedup vs baseline: 2.4792x; 1.0176x over previous
import functools

import jax
import jax.numpy as jnp
import numpy as np
from jax.experimental import pallas as pl
from jax.experimental.pallas import tpu as pltpu

_STEPS = 5
_ROTS = ((13, 15, 26, 6), (17, 29, 16, 24))
_BLOCK_ROWS = 400
_CW = 1024


def _col_chunks(w):
    chunks = [(c, _CW) for c in range(0, w - _CW + 1, _CW)]
    done = len(chunks) * _CW
    if done < w:
        chunks.append((done, w - done))
    return chunks


def _threefry_bits(x1, seed):
    ks1 = np.uint32(seed)
    ks2 = np.uint32(np.uint32(seed) ^ np.uint32(0x1BD11BDA))
    ks = (np.uint32(0), ks1, ks2)
    x0 = jnp.zeros_like(x1)
    for i in range(5):
        for d in _ROTS[i % 2]:
            x0 = x0 + x1
            x1 = (x1 << jnp.uint32(d)) | (x1 >> jnp.uint32(32 - d))
            x1 = x1 ^ x0
        x0 = x0 + ks[(i + 1) % 3]
        x1 = x1 + np.uint32((int(ks[(i + 2) % 3]) + i + 1) & 0xFFFFFFFF)
    return x0 ^ x1


def _bits_to_float12(bits):
    return jax.lax.bitcast_convert_type(
        (bits >> jnp.uint32(9)) | jnp.uint32(0x3F800000), jnp.float32)


def _schedules(sp):
    gamma_start = 0.1 * (1.0 - sp) + 0.001
    gamma_end = gamma_start * 0.1
    eps_start = 0.005 * sp + 0.0001
    eps_end = eps_start * 0.1
    g_cum, e_cum = [], []
    pg = jnp.float32(1.0)
    pe = jnp.float32(1.0)
    for s in range(_STEPS):
        frac = jnp.float32(s / (_STEPS - 1))
        g = gamma_start + (gamma_end - gamma_start) * frac
        ep = jnp.minimum(eps_start + (eps_end - eps_start) * frac,
                         jnp.float32(0.01))
        pg = pg * (1.0 - g)
        pe = pe * (1.0 - ep)
        g_cum.append(1.0 - pg)
        e_cum.append(1.0 - pe)
    return g_cum, e_cum


def _count_zero_body(xa_ref, xb_ref, o_ref):
    br, w = xa_ref.shape
    chunks = _col_chunks(w)
    groups = br // 8
    unroll = 5 if groups % 5 == 0 else (4 if groups % 4 == 0 else 1)

    def row_groups(gi, carry):
        accs = list(carry)
        for u in range(unroll):
            rs = (gi * unroll + u) * 8
            for k, (cs, cw) in enumerate(chunks):
                accs[2 * k] = accs[2 * k] + xa_ref[pl.ds(rs, 8),
                                                   pl.ds(cs, cw)]
                accs[2 * k + 1] = accs[2 * k + 1] + xb_ref[pl.ds(rs, 8),
                                                           pl.ds(cs, cw)]
        return tuple(accs)

    init = tuple(jnp.zeros((8, cw), jnp.float32)
                 for _, cw in chunks for _ in range(2))
    accs = jax.lax.fori_loop(0, groups // unroll, row_groups, init)
    ones = jnp.float32(0.0)
    for acc in accs:
        ones += jnp.sum(acc)
    total = jnp.float32(2 * br * w) - ones

    @pl.when(pl.program_id(0) == 0)
    def _init():
        o_ref[0, 0] = jnp.float32(0.0)

    o_ref[0, 0] += total


def _flip_chunk(x, x1a, x1b, a0c, a1c):
    f_noise = _bits_to_float12(_threefry_bits(x1a, 1234))
    u_bern = _bits_to_float12(_threefry_bits(x1b, 5678)) - 1.0
    a1 = jnp.where(x == 0.0, a0c, a1c)
    prob = jax.nn.sigmoid(a1 - f_noise)
    flip = u_bern < prob
    return jnp.where(flip, 1.0 - x, x)


def _flip_body(sp_ref, t_ref, pa_ref, pb_ref, x_ref, o_ref, *, n_cols):
    sp = sp_ref[0, 0]
    g_cum, e_cum = _schedules(sp)

    br, w = x_ref.shape
    chunks = _col_chunks(w)
    col0 = jnp.uint32(pl.program_id(0) * br)

    tc = t_ref[0:1, :]
    a0v = jnp.ones_like(tc, dtype=jnp.float32)
    a1v = a0v
    for s in range(_STEPS):
        sel = tc == s
        a0v = jnp.where(sel, g_cum[s] + 1.0, a0v)
        a1v = jnp.where(sel, e_cum[s] + 1.0, a1v)

    groups = br // 8
    unroll = 5 if groups % 5 == 0 else (2 if groups % 2 == 0 else 1)

    def row_groups(gi, _):
        for u in range(unroll):
            rs = (gi * unroll + u) * 8
            cbase = col0 + jnp.uint32(rs)
            for cs, cw in chunks:
                x = x_ref[pl.ds(rs, 8), pl.ds(cs, cw)]
                x1a = pa_ref[:, pl.ds(cs, cw)] + cbase
                x1b = pb_ref[:, pl.ds(cs, cw)] + cbase
                o_ref[pl.ds(rs, 8), pl.ds(cs, cw)] = _flip_chunk(
                    x, x1a, x1b, a0v[:, cs:cs + cw], a1v[:, cs:cs + cw])
        return 0

    jax.lax.fori_loop(0, groups // unroll, row_groups, 0)


@jax.jit
def kernel(x_start, t):
    n_rows, n_cols = x_start.shape
    xt = x_start.T

    brt = min(_BLOCK_ROWS, n_cols)
    grid = n_cols // brt

    cbr = brt // 2
    cgrid = n_cols // (2 * cbr)
    count0 = pl.pallas_call(
        _count_zero_body,
        grid=(cgrid,),
        in_specs=[
            pl.BlockSpec((cbr, n_rows), lambda i: (i, 0)),
            pl.BlockSpec((cbr, n_rows), lambda i, _g=cgrid: (i + _g, 0)),
        ],
        out_specs=pl.BlockSpec(memory_space=pltpu.SMEM),
        out_shape=jax.ShapeDtypeStruct((1, 1), jnp.float32),
    )(xt, xt)

    sparsity = count0 / jnp.float32(n_rows * n_cols)

    t2 = t.reshape(1, n_rows)
    pat = (jnp.arange(n_rows, dtype=jnp.uint32)[None, :] * jnp.uint32(n_cols)
           + jnp.arange(8, dtype=jnp.uint32)[:, None])
    pa = pat + jnp.uint32(1234)
    pb = pat + jnp.uint32(5678)
    out_t = pl.pallas_call(
        functools.partial(_flip_body, n_cols=n_cols),
        grid=(grid,),
        in_specs=[
            pl.BlockSpec(memory_space=pltpu.SMEM),
            pl.BlockSpec((1, n_rows), lambda i: (0, 0)),
            pl.BlockSpec((8, n_rows), lambda i: (0, 0)),
            pl.BlockSpec((8, n_rows), lambda i: (0, 0)),
            pl.BlockSpec((brt, n_rows), lambda i: (i, 0)),
        ],
        out_specs=pl.BlockSpec((brt, n_rows), lambda i: (i, 0)),
        out_shape=jax.ShapeDtypeStruct((n_cols, n_rows), jnp.float32),
        compiler_params=pltpu.CompilerParams(
            dimension_semantics=("parallel",)),
    )(sparsity, t2, pa, pb, xt)
    return out_t.T


# speedup vs baseline: 2.4890x; 1.0040x over previous
import functools

import jax
import jax.numpy as jnp
import numpy as np
from jax.experimental import pallas as pl
from jax.experimental.pallas import tpu as pltpu

_STEPS = 5
_ROTS = ((13, 15, 26, 6), (17, 29, 16, 24))
_BLOCK_ROWS = 400
_CW = 1024


def _col_chunks(w):
    chunks = [(c, _CW) for c in range(0, w - _CW + 1, _CW)]
    done = len(chunks) * _CW
    if done < w:
        chunks.append((done, w - done))
    return chunks


def _threefry_bits(x1, seed):
    ks1 = np.uint32(seed)
    ks2 = np.uint32(np.uint32(seed) ^ np.uint32(0x1BD11BDA))
    ks = (np.uint32(0), ks1, ks2)
    x0 = jnp.zeros_like(x1)
    for i in range(5):
        for d in _ROTS[i % 2]:
            x0 = x0 + x1
            x1 = (x1 << jnp.uint32(d)) | (x1 >> jnp.uint32(32 - d))
            x1 = x1 ^ x0
        x0 = x0 + ks[(i + 1) % 3]
        x1 = x1 + np.uint32((int(ks[(i + 2) % 3]) + i + 1) & 0xFFFFFFFF)
    return x0 ^ x1


def _bits_to_float12(bits):
    return jax.lax.bitcast_convert_type(
        (bits >> jnp.uint32(9)) | jnp.uint32(0x3F800000), jnp.float32)


def _schedules(sp):
    gamma_start = 0.1 * (1.0 - sp) + 0.001
    gamma_end = gamma_start * 0.1
    eps_start = 0.005 * sp + 0.0001
    eps_end = eps_start * 0.1
    g_cum, e_cum = [], []
    pg = jnp.float32(1.0)
    pe = jnp.float32(1.0)
    for s in range(_STEPS):
        frac = jnp.float32(s / (_STEPS - 1))
        g = gamma_start + (gamma_end - gamma_start) * frac
        ep = jnp.minimum(eps_start + (eps_end - eps_start) * frac,
                         jnp.float32(0.01))
        pg = pg * (1.0 - g)
        pe = pe * (1.0 - ep)
        g_cum.append(1.0 - pg)
        e_cum.append(1.0 - pe)
    return g_cum, e_cum


def _count_zero_body(xa_ref, xb_ref, o_ref):
    br, w = xa_ref.shape
    chunks = _col_chunks(w)
    groups = br // 8
    unroll = 5 if groups % 5 == 0 else (4 if groups % 4 == 0 else 1)

    def row_groups(gi, carry):
        accs = list(carry)
        for u in range(unroll):
            rs = (gi * unroll + u) * 8
            for k, (cs, cw) in enumerate(chunks):
                accs[2 * k] = accs[2 * k] + xa_ref[pl.ds(rs, 8),
                                                   pl.ds(cs, cw)]
                accs[2 * k + 1] = accs[2 * k + 1] + xb_ref[pl.ds(rs, 8),
                                                           pl.ds(cs, cw)]
        return tuple(accs)

    init = tuple(jnp.zeros((8, cw), jnp.float32)
                 for _, cw in chunks for _ in range(2))
    accs = jax.lax.fori_loop(0, groups // unroll, row_groups, init)
    ones = jnp.float32(0.0)
    for acc in accs:
        ones += jnp.sum(acc)
    total = jnp.float32(2 * br * w) - ones

    @pl.when(pl.program_id(0) == 0)
    def _init():
        o_ref[0, 0] = jnp.float32(0.0)

    o_ref[0, 0] += total


def _flip_chunk(x, x1a, x1b, a0c, a1c):
    f_noise = _bits_to_float12(_threefry_bits(x1a, 1234))
    u_bern = _bits_to_float12(_threefry_bits(x1b, 5678)) - 1.0
    a1 = jnp.where(x == 0.0, a0c, a1c)
    prob = jax.nn.sigmoid(a1 - f_noise)
    flip = u_bern < prob
    return jnp.where(flip, 1.0 - x, x)


def _flip_body(sp_ref, t_ref, pa_ref, pb_ref, x_ref, o_ref, *, n_cols):
    sp = sp_ref[0, 0]
    g_cum, e_cum = _schedules(sp)

    br, w = x_ref.shape
    chunks = _col_chunks(w)
    col0 = jnp.uint32(pl.program_id(0) * br)

    tc = t_ref[0:1, :]
    a0v = jnp.ones_like(tc, dtype=jnp.float32)
    a1v = a0v
    for s in range(_STEPS):
        sel = tc == s
        a0v = jnp.where(sel, g_cum[s] + 1.0, a0v)
        a1v = jnp.where(sel, e_cum[s] + 1.0, a1v)

    groups = br // 8
    unroll = 10 if groups % 10 == 0 else (5 if groups % 5 == 0 else 1)

    def row_groups(gi, _):
        for u in range(unroll):
            rs = (gi * unroll + u) * 8
            cbase = col0 + jnp.uint32(rs)
            for cs, cw in chunks:
                x = x_ref[pl.ds(rs, 8), pl.ds(cs, cw)]
                x1a = pa_ref[:, pl.ds(cs, cw)] + cbase
                x1b = pb_ref[:, pl.ds(cs, cw)] + cbase
                o_ref[pl.ds(rs, 8), pl.ds(cs, cw)] = _flip_chunk(
                    x, x1a, x1b, a0v[:, cs:cs + cw], a1v[:, cs:cs + cw])
        return 0

    jax.lax.fori_loop(0, groups // unroll, row_groups, 0)


@jax.jit
def kernel(x_start, t):
    n_rows, n_cols = x_start.shape
    xt = x_start.T

    brt = min(_BLOCK_ROWS, n_cols)
    grid = n_cols // brt

    cbr = brt // 2
    cgrid = n_cols // (2 * cbr)
    count0 = pl.pallas_call(
        _count_zero_body,
        grid=(cgrid,),
        in_specs=[
            pl.BlockSpec((cbr, n_rows), lambda i: (i, 0)),
            pl.BlockSpec((cbr, n_rows), lambda i, _g=cgrid: (i + _g, 0)),
        ],
        out_specs=pl.BlockSpec(memory_space=pltpu.SMEM),
        out_shape=jax.ShapeDtypeStruct((1, 1), jnp.float32),
    )(xt, xt)

    sparsity = count0 / jnp.float32(n_rows * n_cols)

    t2 = t.reshape(1, n_rows)
    pat = (jnp.arange(n_rows, dtype=jnp.uint32)[None, :] * jnp.uint32(n_cols)
           + jnp.arange(8, dtype=jnp.uint32)[:, None])
    pa = pat + jnp.uint32(1234)
    pb = pat + jnp.uint32(5678)
    out_t = pl.pallas_call(
        functools.partial(_flip_body, n_cols=n_cols),
        grid=(grid,),
        in_specs=[
            pl.BlockSpec(memory_space=pltpu.SMEM),
            pl.BlockSpec((1, n_rows), lambda i: (0, 0)),
            pl.BlockSpec((8, n_rows), lambda i: (0, 0)),
            pl.BlockSpec((8, n_rows), lambda i: (0, 0)),
            pl.BlockSpec((brt, n_rows), lambda i: (i, 0)),
        ],
        out_specs=pl.BlockSpec((brt, n_rows), lambda i: (i, 0)),
        out_shape=jax.ShapeDtypeStruct((n_cols, n_rows), jnp.float32),
        compiler_params=pltpu.CompilerParams(
            dimension_semantics=("parallel",)),
    )(sparsity, t2, pa, pb, xt)
    return out_t.T
